# Initial kernel scaffold; baseline (speedup 1.0000x reference)
import jax
import jax.numpy as jnp
from jax.experimental import pallas as pl


def kernel(x, edge_index, W_l, b_l, W_r):
    raise NotImplementedError("write your pallas kernel here")


# speedup vs baseline: 5.7682x; 5.7682x over previous
import functools

import jax
import jax.numpy as jnp
from jax import lax
from jax.experimental import pallas as pl
from jax.experimental.pallas import tpu as pltpu
from jax.experimental.pallas import tpu_sc as plsc

N_NODES = 10000
D = 128
N_EDGES = 320000

NW = 32
CHUNK = 128
CPW = 80
EPW = CHUNK * CPW
E_PAD = NW * EPW
N_PAD = 10240
DUMMY = N_PAD - 1
ROWS_PER_TILE = N_PAD // 16

TC_BLK = 2048


def _sc_body(x_hbm, src_hbm, dst_hbm, parts_out, cnt_out,
             src_l, dst_l, rows0, cnt_l, iota80, acc, cnt_sh, sem0):
    c_id = lax.axis_index("c")
    s_id = lax.axis_index("s")
    wid = s_id * 2 + c_id

    pltpu.sync_copy(src_hbm.at[pl.ds(wid * CPW, CPW)], src_l)
    pltpu.sync_copy(dst_hbm.at[pl.ds(wid * CPW, CPW)], dst_l)

    zeros16 = jnp.zeros((16,), jnp.float32)

    def zrow(j, carry):
        for l in range(8):
            rows0[j, pl.ds(l * 16, 16)] = zeros16
        return carry

    lax.fori_loop(0, CHUNK, zrow, 0)

    def zcnt(j, carry):
        for l in range(8):
            cnt_l[j, pl.ds(l * 16, 16)] = zeros16
        return carry

    lax.fori_loop(0, CPW, zcnt, 0)

    for i in range(CPW // 16):
        iota80[pl.ds(i * 16, 16)] = lax.iota(jnp.int32, 16) + (i * 16)

    base = s_id * ROWS_PER_TILE
    for k in range(ROWS_PER_TILE // CHUNK):
        pltpu.sync_copy(rows0, acc.at[pl.ds(base + k * CHUNK, CHUNK)])

    @pl.when(s_id == 0)
    def _():
        pltpu.sync_copy(rows0.at[pl.ds(0, CPW)], cnt_sh)

    plsc.subcore_barrier()

    ones16 = jnp.full((16,), 1.0, jnp.float32)

    def chunk_body(c, carry):
        pltpu.async_copy(x_hbm.at[src_l.at[c]], rows0, sem0).wait()
        pltpu.sync_copy(rows0, acc.at[dst_l.at[c]], add=True)
        for l in range(8):
            d = dst_l[c, pl.ds(l * 16, 16)]
            hi = lax.shift_right_logical(d, 7)
            lo = lax.bitwise_and(d, 127)
            plsc.addupdate_scatter(cnt_l, [hi, lo], ones16)
        return carry

    lax.fori_loop(0, CPW, chunk_body, 0)

    pltpu.sync_copy(cnt_l, cnt_sh.at[iota80], add=True)

    plsc.subcore_barrier()

    pltpu.sync_copy(acc.at[pl.ds(base, ROWS_PER_TILE)],
                    parts_out.at[pl.ds(c_id * N_PAD + base, ROWS_PER_TILE)])

    @pl.when(s_id == 0)
    def _():
        pltpu.sync_copy(cnt_sh, cnt_out.at[pl.ds(c_id * CPW, CPW)])


_sc_call = functools.partial(
    pl.kernel,
    out_type=[
        jax.ShapeDtypeStruct((2 * N_PAD, D), jnp.float32),
        jax.ShapeDtypeStruct((2 * CPW, CHUNK), jnp.float32),
    ],
    mesh=plsc.VectorSubcoreMesh(core_axis_name="c", subcore_axis_name="s"),
    compiler_params=pltpu.CompilerParams(needs_layout_passes=False),
    scratch_types=[
        pltpu.VMEM((CPW, CHUNK), jnp.int32),
        pltpu.VMEM((CPW, CHUNK), jnp.int32),
        pltpu.VMEM((CHUNK, D), jnp.float32),
        pltpu.VMEM((CPW, CHUNK), jnp.float32),
        pltpu.VMEM((CPW,), jnp.int32),
        pltpu.VMEM_SHARED((N_PAD, D), jnp.float32),
        pltpu.VMEM_SHARED((CPW, CHUNK), jnp.float32),
        pltpu.SemaphoreType.DMA,
    ],
)


def _tc_body(x_ref, p0_ref, p1_ref, c0_ref, c1_ref, wl_ref, wr_ref, bl_ref,
             o_ref):
    s = x_ref[...] + p0_ref[...] + p1_ref[...]
    inv = 1.0 / (c0_ref[...] + c1_ref[...] + 1.0)
    h = jnp.dot(s, wl_ref[...], preferred_element_type=jnp.float32) * inv
    h = h + bl_ref[...] + jnp.dot(x_ref[...], wr_ref[...],
                                  preferred_element_type=jnp.float32)
    o_ref[...] = jnp.maximum(h, 0.0)


def kernel(x, edge_index, W_l, b_l, W_r):
    src = edge_index[0].astype(jnp.int32)
    dst = edge_index[1].astype(jnp.int32)
    pad = E_PAD - N_EDGES
    src_p = jnp.concatenate([src, jnp.zeros((pad,), jnp.int32)])
    dst_p = jnp.concatenate([dst, jnp.full((pad,), DUMMY, jnp.int32)])
    src_p = src_p.reshape(NW * CPW, CHUNK)
    dst_p = dst_p.reshape(NW * CPW, CHUNK)

    parts, cnts = _sc_call(_sc_body)(x, src_p, dst_p)

    c0 = cnts[:CPW].reshape(N_PAD, 1)
    c1 = cnts[CPW:].reshape(N_PAD, 1)
    wl = W_l.T
    wr = W_r.T
    bl = b_l.reshape(1, D)

    nb = N_PAD // TC_BLK
    out = pl.pallas_call(
        _tc_body,
        grid=(nb,),
        in_specs=[
            pl.BlockSpec((TC_BLK, D), lambda i: (i, 0)),
            pl.BlockSpec((TC_BLK, D), lambda i: (i, 0)),
            pl.BlockSpec((TC_BLK, D), lambda i: (i + nb, 0)),
            pl.BlockSpec((TC_BLK, 1), lambda i: (i, 0)),
            pl.BlockSpec((TC_BLK, 1), lambda i: (i, 0)),
            pl.BlockSpec((D, D), lambda i: (0, 0)),
            pl.BlockSpec((D, D), lambda i: (0, 0)),
            pl.BlockSpec((1, D), lambda i: (0, 0)),
        ],
        out_specs=pl.BlockSpec((TC_BLK, D), lambda i: (i, 0)),
        out_shape=jax.ShapeDtypeStruct((N_NODES, D), jnp.float32),
    )(x, parts, parts, c0, c1, wl, wr, bl)
    return out


# speedup vs baseline: 6.7897x; 1.1771x over previous
import functools

import jax
import jax.numpy as jnp
from jax import lax
from jax.experimental import pallas as pl
from jax.experimental.pallas import tpu as pltpu
from jax.experimental.pallas import tpu_sc as plsc

N_NODES = 10000
D = 128
N_EDGES = 320000

NW = 32
CHUNK = 128
CPW = 80
SC_CH = 8
NSC = CPW // SC_CH
CGRID = 80
EPW = CHUNK * CPW
E_PAD = NW * EPW
N_PAD = 10240
DUMMY = N_PAD - 1
ROWS_PER_TILE = N_PAD // 16

TC_BLK = 2048


def _sc_body(x_hbm, ed_hbm, parts_out, cnt_out,
             idx_l, rows0, rows1, cnt_l, iota80, acc, cnt_sh,
             sem_i0, sem_i1, sem_g0, sem_g1):
    c_id = lax.axis_index("c")
    s_id = lax.axis_index("s")
    wid = s_id * 2 + c_id
    wbase = wid * CPW

    zeros16 = jnp.zeros((16,), jnp.float32)

    def zrow(j, carry):
        for l in range(8):
            rows0[j, pl.ds(l * 16, 16)] = zeros16
        return carry

    lax.fori_loop(0, CHUNK, zrow, 0)

    def zcnt(j, carry):
        for l in range(8):
            cnt_l[j, pl.ds(l * 16, 16)] = zeros16
        return carry

    lax.fori_loop(0, CGRID, zcnt, 0)

    for i in range(CGRID // 16):
        iota80[pl.ds(i * 16, 16)] = lax.iota(jnp.int32, 16) + (i * 16)

    base = s_id * ROWS_PER_TILE
    for k in range(ROWS_PER_TILE // CHUNK):
        pltpu.sync_copy(rows0, acc.at[pl.ds(base + k * CHUNK, CHUNK)])

    @pl.when(s_id == 0)
    def _():
        pltpu.sync_copy(cnt_l, cnt_sh)

    plsc.subcore_barrier()

    ones16 = jnp.full((16,), 1.0, jnp.float32)
    bufs = (rows0, rows1)
    sem_i = (sem_i0, sem_i1)
    sem_g = (sem_g0, sem_g1)

    pltpu.async_copy(ed_hbm.at[pl.ds(wbase, SC_CH)], idx_l.at[0], sem_i0)
    pltpu.async_copy(ed_hbm.at[pl.ds(wbase + SC_CH, SC_CH)], idx_l.at[1],
                     sem_i1)
    pltpu.make_async_copy(ed_hbm.at[pl.ds(wbase, SC_CH)], idx_l.at[0],
                          sem_i0).wait()
    pltpu.async_copy(x_hbm.at[idx_l.at[0, 0, 0]], rows0, sem_g0)
    pltpu.async_copy(x_hbm.at[idx_l.at[0, 1, 0]], rows1, sem_g1)

    def super_body(j, carry):
        for p in range(2):
            s = 2 * j + p
            sbase = wbase + s * SC_CH

            @pl.when(s + 1 < NSC)
            def _():
                pltpu.make_async_copy(
                    ed_hbm.at[pl.ds(sbase + SC_CH, SC_CH)],
                    idx_l.at[1 - p], sem_i[1 - p]).wait()

            for k in range(SC_CH):
                b = k % 2

                pltpu.make_async_copy(x_hbm.at[idx_l.at[p, k, 0]], bufs[b],
                                      sem_g[b]).wait()
                pltpu.sync_copy(bufs[b], acc.at[idx_l.at[p, k, 1]], add=True)

                if k < SC_CH - 2:
                    pltpu.async_copy(x_hbm.at[idx_l.at[p, k + 2, 0]],
                                     bufs[b], sem_g[b])
                else:
                    kk = k - (SC_CH - 2)

                    @pl.when(s + 1 < NSC)
                    def _(kk=kk, b=b):
                        pltpu.async_copy(x_hbm.at[idx_l.at[1 - p, kk, 0]],
                                         bufs[b], sem_g[b])

                for l in range(8):
                    d = idx_l[p, k, 1, pl.ds(l * 16, 16)]
                    hi = lax.shift_right_logical(d, 7)
                    lo = lax.bitwise_and(d, 127)
                    plsc.addupdate_scatter(cnt_l, [hi, lo], ones16)

            @pl.when(s + 2 < NSC)
            def _():
                pltpu.async_copy(ed_hbm.at[pl.ds(sbase + 2 * SC_CH, SC_CH)],
                                 idx_l.at[p], sem_i[p])

        return carry

    lax.fori_loop(0, NSC // 2, super_body, 0)

    pltpu.sync_copy(cnt_l, cnt_sh.at[iota80], add=True)

    plsc.subcore_barrier()

    pltpu.sync_copy(acc.at[pl.ds(base, ROWS_PER_TILE)],
                    parts_out.at[pl.ds(c_id * N_PAD + base, ROWS_PER_TILE)])

    @pl.when(s_id == 0)
    def _():
        pltpu.sync_copy(cnt_sh, cnt_out.at[pl.ds(c_id * CGRID, CGRID)])


_sc_call = functools.partial(
    pl.kernel,
    out_type=[
        jax.ShapeDtypeStruct((2 * N_PAD, D), jnp.float32),
        jax.ShapeDtypeStruct((2 * CGRID, 128), jnp.float32),
    ],
    mesh=plsc.VectorSubcoreMesh(core_axis_name="c", subcore_axis_name="s"),
    compiler_params=pltpu.CompilerParams(needs_layout_passes=False),
    scratch_types=[
        pltpu.VMEM((2, SC_CH, 2, CHUNK), jnp.int32),
        pltpu.VMEM((CHUNK, D), jnp.float32),
        pltpu.VMEM((CHUNK, D), jnp.float32),
        pltpu.VMEM((CGRID, 128), jnp.float32),
        pltpu.VMEM((CGRID,), jnp.int32),
        pltpu.VMEM_SHARED((N_PAD, D), jnp.float32),
        pltpu.VMEM_SHARED((CGRID, 128), jnp.float32),
        pltpu.SemaphoreType.DMA,
        pltpu.SemaphoreType.DMA,
        pltpu.SemaphoreType.DMA,
        pltpu.SemaphoreType.DMA,
    ],
)


def _tc_body(x_ref, p0_ref, p1_ref, c0_ref, c1_ref, wl_ref, wr_ref, bl_ref,
             o_ref):
    s = x_ref[...] + p0_ref[...] + p1_ref[...]
    inv = 1.0 / (c0_ref[...] + c1_ref[...] + 1.0)
    h = jnp.dot(s, wl_ref[...], preferred_element_type=jnp.float32) * inv
    h = h + bl_ref[...] + jnp.dot(x_ref[...], wr_ref[...],
                                  preferred_element_type=jnp.float32)
    o_ref[...] = jnp.maximum(h, 0.0)


def kernel(x, edge_index, W_l, b_l, W_r):
    src = edge_index[0].astype(jnp.int32)
    dst = edge_index[1].astype(jnp.int32)
    pad = E_PAD - N_EDGES
    src_p = jnp.concatenate([src, jnp.zeros((pad,), jnp.int32)])
    dst_p = jnp.concatenate([dst, jnp.full((pad,), DUMMY, jnp.int32)])
    ed_p = jnp.concatenate(
        [src_p.reshape(NW * CPW, 1, CHUNK), dst_p.reshape(NW * CPW, 1, CHUNK)],
        axis=1)

    parts, cnts = _sc_call(_sc_body)(x, ed_p)

    c0 = cnts[:CGRID].reshape(N_PAD, 1)
    c1 = cnts[CGRID:].reshape(N_PAD, 1)
    wl = W_l.T
    wr = W_r.T
    bl = b_l.reshape(1, D)

    nb = N_PAD // TC_BLK
    out = pl.pallas_call(
        _tc_body,
        grid=(nb,),
        in_specs=[
            pl.BlockSpec((TC_BLK, D), lambda i: (i, 0)),
            pl.BlockSpec((TC_BLK, D), lambda i: (i, 0)),
            pl.BlockSpec((TC_BLK, D), lambda i: (i + nb, 0)),
            pl.BlockSpec((TC_BLK, 1), lambda i: (i, 0)),
            pl.BlockSpec((TC_BLK, 1), lambda i: (i, 0)),
            pl.BlockSpec((D, D), lambda i: (0, 0)),
            pl.BlockSpec((D, D), lambda i: (0, 0)),
            pl.BlockSpec((1, D), lambda i: (0, 0)),
        ],
        out_specs=pl.BlockSpec((TC_BLK, D), lambda i: (i, 0)),
        out_shape=jax.ShapeDtypeStruct((N_NODES, D), jnp.float32),
    )(x, parts, parts, c0, c1, wl, wr, bl)
    return out


# speedup vs baseline: 6.7963x; 1.0010x over previous
import functools

import jax
import jax.numpy as jnp
from jax import lax
from jax.experimental import pallas as pl
from jax.experimental.pallas import tpu as pltpu
from jax.experimental.pallas import tpu_sc as plsc

N_NODES = 10000
D = 128
N_EDGES = 320000

NW = 32
CHUNK = 128
CPW = 80
SC_CH = 8
NSC = CPW // SC_CH
CGRID = 80
EPW = CHUNK * CPW
E_PAD = NW * EPW
N_PAD = 10240
DUMMY = N_PAD - 1
ROWS_PER_TILE = N_PAD // 16

TC_BLK = 2048


def _sc_body(x_hbm, ed_hbm, parts_out, cnt_out,
             idx_l, rows0, rows1, cnt_l, iota80, acc, cnt_sh,
             sem_i0, sem_i1, sem_g0, sem_g1):
    c_id = lax.axis_index("c")
    s_id = lax.axis_index("s")
    wid = s_id * 2 + c_id
    wbase = wid * CPW

    zeros16 = jnp.zeros((16,), jnp.float32)

    def zrow(j, carry):
        for l in range(8):
            rows0[j, pl.ds(l * 16, 16)] = zeros16
        return carry

    lax.fori_loop(0, CHUNK, zrow, 0)

    def zcnt(j, carry):
        for l in range(8):
            cnt_l[j, pl.ds(l * 16, 16)] = zeros16
        return carry

    lax.fori_loop(0, CGRID, zcnt, 0)

    for i in range(CGRID // 16):
        iota80[pl.ds(i * 16, 16)] = lax.iota(jnp.int32, 16) + (i * 16)

    base = s_id * ROWS_PER_TILE
    for k in range(ROWS_PER_TILE // CHUNK):
        pltpu.sync_copy(rows0, acc.at[pl.ds(base + k * CHUNK, CHUNK)])

    @pl.when(s_id == 0)
    def _():
        pltpu.sync_copy(cnt_l, cnt_sh)

    plsc.subcore_barrier()

    ones16 = jnp.full((16,), 1.0, jnp.float32)
    bufs = (rows0, rows1)
    sem_i = (sem_i0, sem_i1)
    sem_g = (sem_g0, sem_g1)

    pltpu.async_copy(ed_hbm.at[pl.ds(wbase, SC_CH)], idx_l.at[0], sem_i0)
    pltpu.async_copy(ed_hbm.at[pl.ds(wbase + SC_CH, SC_CH)], idx_l.at[1],
                     sem_i1)
    pltpu.make_async_copy(ed_hbm.at[pl.ds(wbase, SC_CH)], idx_l.at[0],
                          sem_i0).wait()
    pltpu.async_copy(x_hbm.at[idx_l.at[0, 0, 0]], rows0, sem_g0)
    pltpu.async_copy(x_hbm.at[idx_l.at[0, 1, 0]], rows1, sem_g1)

    def super_body(j, carry):
        for p in range(2):
            s = 2 * j + p
            sbase = wbase + s * SC_CH

            @pl.when(s + 1 < NSC)
            def _():
                pltpu.make_async_copy(
                    ed_hbm.at[pl.ds(sbase + SC_CH, SC_CH)],
                    idx_l.at[1 - p], sem_i[1 - p]).wait()

            for k in range(SC_CH):
                b = k % 2

                pltpu.make_async_copy(x_hbm.at[idx_l.at[p, k, 0]], bufs[b],
                                      sem_g[b]).wait()
                pltpu.sync_copy(bufs[b], acc.at[idx_l.at[p, k, 1]], add=True)

                if k < SC_CH - 2:
                    pltpu.async_copy(x_hbm.at[idx_l.at[p, k + 2, 0]],
                                     bufs[b], sem_g[b])
                else:
                    kk = k - (SC_CH - 2)

                    @pl.when(s + 1 < NSC)
                    def _(kk=kk, b=b):
                        pltpu.async_copy(x_hbm.at[idx_l.at[1 - p, kk, 0]],
                                         bufs[b], sem_g[b])

                for l in range(8):
                    d = idx_l[p, k, 1, pl.ds(l * 16, 16)]
                    hi = lax.shift_right_logical(d, 7)
                    lo = lax.bitwise_and(d, 127)
                    plsc.addupdate_scatter(cnt_l, [hi, lo], ones16)

            @pl.when(s + 2 < NSC)
            def _():
                pltpu.async_copy(ed_hbm.at[pl.ds(sbase + 2 * SC_CH, SC_CH)],
                                 idx_l.at[p], sem_i[p])

        return carry

    lax.fori_loop(0, NSC // 2, super_body, 0)

    pltpu.sync_copy(cnt_l, cnt_sh.at[iota80], add=True)

    plsc.subcore_barrier()

    pltpu.sync_copy(acc.at[pl.ds(base, ROWS_PER_TILE)],
                    parts_out.at[pl.ds(c_id * N_PAD + base, ROWS_PER_TILE)])

    @pl.when(s_id == 0)
    def _():
        pltpu.sync_copy(cnt_sh, cnt_out.at[pl.ds(c_id * CGRID, CGRID)])


_sc_call = functools.partial(
    pl.kernel,
    out_type=[
        jax.ShapeDtypeStruct((2 * N_PAD, D), jnp.float32),
        jax.ShapeDtypeStruct((2 * CGRID, 128), jnp.float32),
    ],
    mesh=plsc.VectorSubcoreMesh(core_axis_name="c", subcore_axis_name="s"),
    compiler_params=pltpu.CompilerParams(needs_layout_passes=False),
    scratch_types=[
        pltpu.VMEM((2, SC_CH, 2, CHUNK), jnp.int32),
        pltpu.VMEM((CHUNK, D), jnp.float32),
        pltpu.VMEM((CHUNK, D), jnp.float32),
        pltpu.VMEM((CGRID, 128), jnp.float32),
        pltpu.VMEM((CGRID,), jnp.int32),
        pltpu.VMEM_SHARED((N_PAD, D), jnp.float32),
        pltpu.VMEM_SHARED((CGRID, 128), jnp.float32),
        pltpu.SemaphoreType.DMA,
        pltpu.SemaphoreType.DMA,
        pltpu.SemaphoreType.DMA,
        pltpu.SemaphoreType.DMA,
    ],
)


def _tc_body(x_ref, p0_ref, p1_ref, c0_ref, c1_ref, wl_ref, wr_ref, bl_ref,
             o_ref):
    s = x_ref[...] + p0_ref[...] + p1_ref[...]
    inv = 1.0 / (c0_ref[...] + c1_ref[...] + 1.0)
    h = jnp.dot(s, wl_ref[...], preferred_element_type=jnp.float32) * inv
    h = h + bl_ref[...] + jnp.dot(x_ref[...], wr_ref[...],
                                  preferred_element_type=jnp.float32)
    o_ref[...] = jnp.maximum(h, 0.0)


def kernel(x, edge_index, W_l, b_l, W_r):
    src = edge_index[0].astype(jnp.int32)
    dst = edge_index[1].astype(jnp.int32)
    pad = E_PAD - N_EDGES
    pad_dst = N_NODES + (jnp.arange(pad, dtype=jnp.int32) % (N_PAD - N_NODES))
    src_p = jnp.concatenate([src, jnp.zeros((pad,), jnp.int32)])
    dst_p = jnp.concatenate([dst, pad_dst])
    ed_p = jnp.concatenate(
        [src_p.reshape(NW * CPW, 1, CHUNK), dst_p.reshape(NW * CPW, 1, CHUNK)],
        axis=1)

    parts, cnts = _sc_call(_sc_body)(x, ed_p)

    c0 = cnts[:CGRID].reshape(N_PAD, 1)
    c1 = cnts[CGRID:].reshape(N_PAD, 1)
    wl = W_l.T
    wr = W_r.T
    bl = b_l.reshape(1, D)

    nb = N_PAD // TC_BLK
    out = pl.pallas_call(
        _tc_body,
        grid=(nb,),
        in_specs=[
            pl.BlockSpec((TC_BLK, D), lambda i: (i, 0)),
            pl.BlockSpec((TC_BLK, D), lambda i: (i, 0)),
            pl.BlockSpec((TC_BLK, D), lambda i: (i + nb, 0)),
            pl.BlockSpec((TC_BLK, 1), lambda i: (i, 0)),
            pl.BlockSpec((TC_BLK, 1), lambda i: (i, 0)),
            pl.BlockSpec((D, D), lambda i: (0, 0)),
            pl.BlockSpec((D, D), lambda i: (0, 0)),
            pl.BlockSpec((1, D), lambda i: (0, 0)),
        ],
        out_specs=pl.BlockSpec((TC_BLK, D), lambda i: (i, 0)),
        out_shape=jax.ShapeDtypeStruct((N_NODES, D), jnp.float32),
    )(x, parts, parts, c0, c1, wl, wr, bl)
    return out


# speedup vs baseline: 6.8863x; 1.0132x over previous
import functools

import jax
import jax.numpy as jnp
from jax import lax
from jax.experimental import pallas as pl
from jax.experimental.pallas import tpu as pltpu
from jax.experimental.pallas import tpu_sc as plsc

N_NODES = 10000
D = 128
N_EDGES = 320000

NW = 32
CHUNK = 128
CPW = 80
CPW0 = 112
CPW1 = 2 * CPW - CPW0
SC_CH = 8
CGRID = 80
EPW = CHUNK * CPW
E_PAD = NW * EPW
N_PAD = 10240
DUMMY = N_PAD - 1
ROWS_PER_TILE = N_PAD // 16

TC_BLK = 2048


def _sc_body(x_hbm, ed_hbm, parts_out, cnt_out,
             idx_l, rows0, rows1, cnt_l, iota80, acc, cnt_sh,
             sem_i0, sem_i1, sem_g0, sem_g1):
    c_id = lax.axis_index("c")
    s_id = lax.axis_index("s")
    is0 = c_id == 0
    wbase = lax.select(is0, s_id * CPW0, 16 * CPW0 + s_id * CPW1)
    nsc = lax.select(is0, CPW0 // SC_CH, CPW1 // SC_CH)

    zeros16 = jnp.zeros((16,), jnp.float32)

    def zrow(j, carry):
        for l in range(8):
            rows0[j, pl.ds(l * 16, 16)] = zeros16
        return carry

    lax.fori_loop(0, CHUNK, zrow, 0)

    def zcnt(j, carry):
        for l in range(8):
            cnt_l[j, pl.ds(l * 16, 16)] = zeros16
        return carry

    lax.fori_loop(0, CGRID, zcnt, 0)

    for i in range(CGRID // 16):
        iota80[pl.ds(i * 16, 16)] = lax.iota(jnp.int32, 16) + (i * 16)

    base = s_id * ROWS_PER_TILE
    for k in range(ROWS_PER_TILE // CHUNK):
        pltpu.sync_copy(rows0, acc.at[pl.ds(base + k * CHUNK, CHUNK)])

    @pl.when(s_id == 0)
    def _():
        pltpu.sync_copy(cnt_l, cnt_sh)

    plsc.subcore_barrier()

    ones16 = jnp.full((16,), 1.0, jnp.float32)
    bufs = (rows0, rows1)
    sem_i = (sem_i0, sem_i1)
    sem_g = (sem_g0, sem_g1)

    pltpu.async_copy(ed_hbm.at[pl.ds(wbase, SC_CH)], idx_l.at[0], sem_i0)
    pltpu.async_copy(ed_hbm.at[pl.ds(wbase + SC_CH, SC_CH)], idx_l.at[1],
                     sem_i1)
    pltpu.make_async_copy(ed_hbm.at[pl.ds(wbase, SC_CH)], idx_l.at[0],
                          sem_i0).wait()
    pltpu.async_copy(x_hbm.at[idx_l.at[0, 0, 0]], rows0, sem_g0)
    pltpu.async_copy(x_hbm.at[idx_l.at[0, 1, 0]], rows1, sem_g1)

    def super_body(j, carry):
        for p in range(2):
            s = 2 * j + p
            sbase = wbase + s * SC_CH

            @pl.when(s + 1 < nsc)
            def _():
                pltpu.make_async_copy(
                    ed_hbm.at[pl.ds(sbase + SC_CH, SC_CH)],
                    idx_l.at[1 - p], sem_i[1 - p]).wait()

            for k in range(SC_CH):
                b = k % 2

                pltpu.make_async_copy(x_hbm.at[idx_l.at[p, k, 0]], bufs[b],
                                      sem_g[b]).wait()
                pltpu.sync_copy(bufs[b], acc.at[idx_l.at[p, k, 1]], add=True)

                if k < SC_CH - 2:
                    pltpu.async_copy(x_hbm.at[idx_l.at[p, k + 2, 0]],
                                     bufs[b], sem_g[b])
                else:
                    kk = k - (SC_CH - 2)

                    @pl.when(s + 1 < nsc)
                    def _(kk=kk, b=b):
                        pltpu.async_copy(x_hbm.at[idx_l.at[1 - p, kk, 0]],
                                         bufs[b], sem_g[b])

                for l in range(8):
                    d = idx_l[p, k, 1, pl.ds(l * 16, 16)]
                    hi = lax.shift_right_logical(d, 7)
                    lo = lax.bitwise_and(d, 127)
                    plsc.addupdate_scatter(cnt_l, [hi, lo], ones16)

            @pl.when(s + 2 < nsc)
            def _():
                pltpu.async_copy(ed_hbm.at[pl.ds(sbase + 2 * SC_CH, SC_CH)],
                                 idx_l.at[p], sem_i[p])

        return carry

    lax.fori_loop(0, nsc // 2, super_body, 0)

    pltpu.sync_copy(cnt_l, cnt_sh.at[iota80], add=True)

    plsc.subcore_barrier()

    pltpu.sync_copy(acc.at[pl.ds(base, ROWS_PER_TILE)],
                    parts_out.at[pl.ds(c_id * N_PAD + base, ROWS_PER_TILE)])

    @pl.when(s_id == 0)
    def _():
        pltpu.sync_copy(cnt_sh, cnt_out.at[pl.ds(c_id * CGRID, CGRID)])


_sc_call = functools.partial(
    pl.kernel,
    out_type=[
        jax.ShapeDtypeStruct((2 * N_PAD, D), jnp.float32),
        jax.ShapeDtypeStruct((2 * CGRID, 128), jnp.float32),
    ],
    mesh=plsc.VectorSubcoreMesh(core_axis_name="c", subcore_axis_name="s"),
    compiler_params=pltpu.CompilerParams(needs_layout_passes=False),
    scratch_types=[
        pltpu.VMEM((2, SC_CH, 2, CHUNK), jnp.int32),
        pltpu.VMEM((CHUNK, D), jnp.float32),
        pltpu.VMEM((CHUNK, D), jnp.float32),
        pltpu.VMEM((CGRID, 128), jnp.float32),
        pltpu.VMEM((CGRID,), jnp.int32),
        pltpu.VMEM_SHARED((N_PAD, D), jnp.float32),
        pltpu.VMEM_SHARED((CGRID, 128), jnp.float32),
        pltpu.SemaphoreType.DMA,
        pltpu.SemaphoreType.DMA,
        pltpu.SemaphoreType.DMA,
        pltpu.SemaphoreType.DMA,
    ],
)


def _tc_body(x_ref, p0_ref, p1_ref, c0_ref, c1_ref, wl_ref, wr_ref, bl_ref,
             o_ref):
    s = x_ref[...] + p0_ref[...] + p1_ref[...]
    inv = 1.0 / (c0_ref[...] + c1_ref[...] + 1.0)
    h = jnp.dot(s, wl_ref[...], preferred_element_type=jnp.float32) * inv
    h = h + bl_ref[...] + jnp.dot(x_ref[...], wr_ref[...],
                                  preferred_element_type=jnp.float32)
    o_ref[...] = jnp.maximum(h, 0.0)


def kernel(x, edge_index, W_l, b_l, W_r):
    src = edge_index[0].astype(jnp.int32)
    dst = edge_index[1].astype(jnp.int32)
    pad = E_PAD - N_EDGES
    pad_dst = N_NODES + (jnp.arange(pad, dtype=jnp.int32) % (N_PAD - N_NODES))
    src_p = jnp.concatenate([src, jnp.zeros((pad,), jnp.int32)])
    dst_p = jnp.concatenate([dst, pad_dst])
    ed_p = jnp.concatenate(
        [src_p.reshape(NW * CPW, 1, CHUNK), dst_p.reshape(NW * CPW, 1, CHUNK)],
        axis=1)

    parts, cnts = _sc_call(_sc_body)(x, ed_p)

    c0 = cnts[:CGRID].reshape(N_PAD, 1)
    c1 = cnts[CGRID:].reshape(N_PAD, 1)
    wl = W_l.T
    wr = W_r.T
    bl = b_l.reshape(1, D)

    nb = N_PAD // TC_BLK
    out = pl.pallas_call(
        _tc_body,
        grid=(nb,),
        in_specs=[
            pl.BlockSpec((TC_BLK, D), lambda i: (i, 0)),
            pl.BlockSpec((TC_BLK, D), lambda i: (i, 0)),
            pl.BlockSpec((TC_BLK, D), lambda i: (i + nb, 0)),
            pl.BlockSpec((TC_BLK, 1), lambda i: (i, 0)),
            pl.BlockSpec((TC_BLK, 1), lambda i: (i, 0)),
            pl.BlockSpec((D, D), lambda i: (0, 0)),
            pl.BlockSpec((D, D), lambda i: (0, 0)),
            pl.BlockSpec((1, D), lambda i: (0, 0)),
        ],
        out_specs=pl.BlockSpec((TC_BLK, D), lambda i: (i, 0)),
        out_shape=jax.ShapeDtypeStruct((N_NODES, D), jnp.float32),
    )(x, parts, parts, c0, c1, wl, wr, bl)
    return out
